# baseline (speedup 1.0000x reference)
import functools

import jax
import jax.numpy as jnp
from jax import lax
from jax.experimental import pallas as pl
from jax.experimental.pallas import tpu as pltpu
from jax.experimental.pallas import tpu_sc as plsc

L, B, T, D = 2, 16, 500, 1024
K = 1000
KP = 1024
KT = 512
BT = B * T
N = BT * L
TB = 125
NT = T // TB
PIECE = 128
NP = KT // PIECE
N4 = N * NP
CH = 128
NCHUNK = N4 // CH
BIG = 3.0e38


def _dist_kernel(x_ref, c_ref, tok_ref):
    x = x_ref[0].reshape(TB * B, D)
    c = c_ref[0]
    xc = lax.dot_general(x, c, (((1,), (1,)), ((), ())),
                         preferred_element_type=jnp.float32)
    x2 = jnp.sum(x * x, axis=1, keepdims=True)
    c2 = jnp.sum(c * c, axis=1)[None, :]
    dist = x2 - 2.0 * xc + c2
    lane = lax.broadcasted_iota(jnp.int32, dist.shape, 1)
    dist = jnp.where(lane < K, dist, BIG)
    m = jnp.min(dist, axis=1, keepdims=True)
    tok_ref[0, 0, :] = jnp.min(jnp.where(dist == m, lane, KP), axis=1)


def _proj_kernel(c_ref, w_ref, b_ref, p_ref):
    c = c_ref[0]
    w = w_ref[0]
    p = lax.dot_general(c, w, (((1,), (1,)), ((), ())),
                        preferred_element_type=jnp.float32)
    p_ref[0] = p + b_ref[0, 0][None, :]


def _tokens(ht, cpad):
    return pl.pallas_call(
        _dist_kernel,
        grid=(L, NT),
        in_specs=[pl.BlockSpec((1, TB, B, D), lambda l, i: (l, i, 0, 0)),
                  pl.BlockSpec((1, KP, D), lambda l, i: (l, 0, 0))],
        out_specs=pl.BlockSpec((1, 1, TB * B), lambda l, i: (l * NT + i, 0, 0)),
        out_shape=jax.ShapeDtypeStruct((L * NT, 1, TB * B), jnp.int32),
    )(ht, cpad)


def _proj_table(cpad, Wst, bst):
    return pl.pallas_call(
        _proj_kernel,
        grid=(L,),
        in_specs=[pl.BlockSpec((1, KP, D), lambda l: (l, 0, 0)),
                  pl.BlockSpec((1, KT, D), lambda l: (l, 0, 0)),
                  pl.BlockSpec((1, 1, KT), lambda l: (l, 0, 0))],
        out_specs=pl.BlockSpec((1, KP, KT), lambda l: (l, 0, 0)),
        out_shape=jax.ShapeDtypeStruct((L, KP, KT), jnp.float32),
    )(cpad, Wst, bst)


def _make_gather():
    info = plsc.get_sparse_core_info()
    NC, NS = info.num_cores, info.num_subcores
    NW = NC * NS
    nloop = -(-NCHUNK // NW)
    mesh = plsc.VectorSubcoreMesh(core_axis_name="c", subcore_axis_name="s")

    @functools.partial(
        pl.kernel, mesh=mesh,
        out_type=jax.ShapeDtypeStruct((N4, PIECE), jnp.float32),
        scratch_types=[
            pltpu.VMEM((CH,), jnp.int32),
            pltpu.VMEM((CH, PIECE), jnp.float32),
            pltpu.SemaphoreType.DMA,
        ],
    )
    def gk(table_hbm, idx_hbm, out_hbm, idx_v, rows_v, sem):
        wid = lax.axis_index("s") * NC + lax.axis_index("c")
        for j in range(nloop):
            c = wid + j * NW

            @pl.when(c < NCHUNK)
            def _():
                off = c * CH
                pltpu.sync_copy(idx_hbm.at[pl.ds(off, CH)], idx_v)
                pltpu.async_copy(table_hbm.at[idx_v], rows_v, sem).wait()
                pltpu.sync_copy(rows_v, out_hbm.at[pl.ds(off, CH)])

    return gk


def kernel(h, centers0, centers1, W0, b0, W1, b1):
    pad = jnp.zeros((KP - K, D), jnp.float32)
    cpad = jnp.stack([jnp.concatenate([centers0, pad], 0),
                      jnp.concatenate([centers1, pad], 0)])
    Wst = jnp.stack([W0, W1])
    bst = jnp.stack([b0, b1]).reshape(L, 1, KT)

    ht = jnp.transpose(h, (0, 2, 1, 3))
    tok3 = _tokens(ht, cpad)
    P = _proj_table(cpad, Wst, bst)

    tokens = tok3.reshape(L, T, B).transpose(2, 1, 0)
    offs = jnp.arange(L, dtype=jnp.int32) * KP
    base4 = ((tokens + offs) * NP)[:, :, None, :]
    idx4 = (base4 + jnp.arange(NP, dtype=jnp.int32)[None, None, :, None])
    idx_flat = idx4.reshape(-1)

    out = _make_gather()(P.reshape(L * KP * NP, PIECE), idx_flat)
    embs = (out.reshape(B, T, NP, L, PIECE)
            .transpose(0, 1, 3, 2, 4).reshape(B, T, L, KT))
    return tokens, embs, tokens


# speedup vs baseline: 1.0155x; 1.0155x over previous
import functools

import jax
import jax.numpy as jnp
from jax import lax
from jax.experimental import pallas as pl
from jax.experimental.pallas import tpu as pltpu
from jax.experimental.pallas import tpu_sc as plsc

L, B, T, D = 2, 16, 500, 1024
K = 1000
KP = 1024
KT = 512
BT = B * T
N = BT * L
TB = 25
NT = T // TB
PIECE = 128
NP = KT // PIECE
N4 = N * NP
CH = 128
NCHUNK = N4 // CH
BIG = 3.0e38


def _dist_kernel(x_ref, c_ref, tok_ref):
    x = x_ref[0].reshape(TB * B, D)
    c = c_ref[0]
    xc = lax.dot_general(x, c, (((1,), (1,)), ((), ())),
                         preferred_element_type=jnp.float32)
    x2 = jnp.sum(x * x, axis=1, keepdims=True)
    c2 = jnp.sum(c * c, axis=1)[None, :]
    dist = x2 - 2.0 * xc + c2
    lane = lax.broadcasted_iota(jnp.int32, dist.shape, 1)
    dist = jnp.where(lane < K, dist, BIG)
    m = jnp.min(dist, axis=1, keepdims=True)
    tok_ref[0, 0, :] = jnp.min(jnp.where(dist == m, lane, KP), axis=1)


def _proj_kernel(c_ref, w_ref, b_ref, p_ref):
    c = c_ref[0]
    w = w_ref[0]
    p = lax.dot_general(c, w, (((1,), (1,)), ((), ())),
                        preferred_element_type=jnp.float32)
    p_ref[0] = p + b_ref[0, 0][None, :]


def _tokens(ht, cpad):
    return pl.pallas_call(
        _dist_kernel,
        grid=(L, NT),
        in_specs=[pl.BlockSpec((1, TB, B, D), lambda l, i: (l, i, 0, 0)),
                  pl.BlockSpec((1, KP, D), lambda l, i: (l, 0, 0))],
        out_specs=pl.BlockSpec((1, 1, TB * B), lambda l, i: (l * NT + i, 0, 0)),
        out_shape=jax.ShapeDtypeStruct((L * NT, 1, TB * B), jnp.int32),
    )(ht, cpad)


def _proj_table(cpad, Wst, bst):
    return pl.pallas_call(
        _proj_kernel,
        grid=(L,),
        in_specs=[pl.BlockSpec((1, KP, D), lambda l: (l, 0, 0)),
                  pl.BlockSpec((1, KT, D), lambda l: (l, 0, 0)),
                  pl.BlockSpec((1, 1, KT), lambda l: (l, 0, 0))],
        out_specs=pl.BlockSpec((1, KP, KT), lambda l: (l, 0, 0)),
        out_shape=jax.ShapeDtypeStruct((L, KP, KT), jnp.float32),
    )(cpad, Wst, bst)


def _make_gather():
    info = plsc.get_sparse_core_info()
    NC, NS = info.num_cores, info.num_subcores
    NW = NC * NS
    nloop = -(-NCHUNK // NW)
    mesh = plsc.VectorSubcoreMesh(core_axis_name="c", subcore_axis_name="s")

    @functools.partial(
        pl.kernel, mesh=mesh,
        out_type=jax.ShapeDtypeStruct((N4, PIECE), jnp.float32),
        scratch_types=[
            pltpu.VMEM((CH,), jnp.int32),
            pltpu.VMEM((CH, PIECE), jnp.float32),
            pltpu.VMEM((CH, PIECE), jnp.float32),
            pltpu.SemaphoreType.DMA,
            pltpu.SemaphoreType.DMA,
        ],
    )
    def gk(table_hbm, idx_hbm, out_hbm, idx_v, rows_v0, rows_v1, gsem, wsem):
        wid = lax.axis_index("s") * NC + lax.axis_index("c")
        rows_s = (rows_v0, rows_v1)
        wbs = [None] * (nloop - 1)
        for j in range(nloop - 1):
            off = (wid + j * NW) * CH
            if j >= 2:
                wbs[j - 2].wait()
            pltpu.sync_copy(idx_hbm.at[pl.ds(off, CH)], idx_v)
            pltpu.async_copy(table_hbm.at[idx_v], rows_s[j % 2], gsem).wait()
            wbs[j] = pltpu.async_copy(
                rows_s[j % 2], out_hbm.at[pl.ds(off, CH)], wsem)
        wbs[nloop - 3].wait()
        wbs[nloop - 2].wait()

        c = wid + (nloop - 1) * NW
        @pl.when(c < NCHUNK)
        def _():
            off = c * CH
            pltpu.sync_copy(idx_hbm.at[pl.ds(off, CH)], idx_v)
            pltpu.async_copy(table_hbm.at[idx_v], rows_v0, gsem).wait()
            pltpu.sync_copy(rows_v0, out_hbm.at[pl.ds(off, CH)])

    return gk


def kernel(h, centers0, centers1, W0, b0, W1, b1):
    pad = jnp.zeros((KP - K, D), jnp.float32)
    cpad = jnp.stack([jnp.concatenate([centers0, pad], 0),
                      jnp.concatenate([centers1, pad], 0)])
    Wst = jnp.stack([W0, W1])
    bst = jnp.stack([b0, b1]).reshape(L, 1, KT)

    ht = jnp.transpose(h, (0, 2, 1, 3))
    tok3 = _tokens(ht, cpad)
    P = _proj_table(cpad, Wst, bst)

    tokens = tok3.reshape(L, T, B).transpose(2, 1, 0)
    offs = jnp.arange(L, dtype=jnp.int32) * KP
    base4 = ((tokens + offs) * NP)[:, :, None, :]
    idx4 = (base4 + jnp.arange(NP, dtype=jnp.int32)[None, None, :, None])
    idx_flat = idx4.reshape(-1)

    out = _make_gather()(P.reshape(L * KP * NP, PIECE), idx_flat)
    embs = (out.reshape(B, T, NP, L, PIECE)
            .transpose(0, 1, 3, 2, 4).reshape(B, T, L, KT))
    return tokens, embs, tokens
